# baseline (speedup 1.0000x reference)
import functools

import jax
import jax.numpy as jnp
from jax import lax
from jax.experimental import pallas as pl
from jax.experimental.pallas import tpu as pltpu
from jax.experimental.pallas import tpu_sc as plsc

N_ATOMS = 32768
HIDDEN = 256
HALF = 128
SEG = 16

NC = 2
NS = 16
NW = NC * NS
CHUNK = N_ATOMS // NW
LANES = 16

BLK = 8192


def _mlp_body(h_ref, w1t_ref, b1_ref, w2_ref, b2_ref, s_ref):
    x = jnp.dot(h_ref[...], w1t_ref[...], preferred_element_type=jnp.float32)
    x = x + b1_ref[...]
    x = x * jax.nn.sigmoid(x)
    s = jnp.sum(x * w2_ref[...], axis=1)
    s_ref[...] = s.reshape(BLK // 128, 128) + b2_ref[...]


def _mlp(h, W1, b1, W2, b2):
    grid = N_ATOMS // BLK
    return pl.pallas_call(
        _mlp_body,
        grid=(grid,),
        in_specs=[
            pl.BlockSpec((BLK, HIDDEN), lambda g: (g, 0)),
            pl.BlockSpec((HIDDEN, HALF), lambda g: (0, 0)),
            pl.BlockSpec((1, HALF), lambda g: (0, 0)),
            pl.BlockSpec((1, HALF), lambda g: (0, 0)),
            pl.BlockSpec((1, 1), lambda g: (0, 0)),
        ],
        out_specs=pl.BlockSpec((BLK // 128, 128), lambda g: (g, 0)),
        out_shape=jax.ShapeDtypeStruct((N_ATOMS // 128, 128), jnp.float32),
        compiler_params=pltpu.CompilerParams(
            dimension_semantics=("arbitrary",),
        ),
    )(h, W1.T, b1.reshape(1, HALF),
      W2.reshape(1, HALF), b2.reshape(1, 1))


ROWS = CHUNK // 128
SPREAD = 128 * SEG


def _segsum_body(s_hbm, b_hbm, out_hbm, val_v, idx_v, idx2_v, fold_v, vec_v,
                 all_v, shared, stage):
    cid = lax.axis_index("c")
    sid = lax.axis_index("s")
    wid = cid * NS + sid
    pltpu.sync_copy(s_hbm.at[pl.ds(wid * ROWS, ROWS)], val_v)
    pltpu.sync_copy(b_hbm.at[pl.ds(wid * CHUNK, CHUNK)], idx_v)

    lane = lax.iota(jnp.int32, LANES)
    base = sid * SPREAD
    for i in range(CHUNK // LANES):
        r, c = divmod(i * LANES, 128)
        d = idx_v[pl.ds(i * LANES, LANES)]
        idx2_v[r, pl.ds(c, LANES)] = d + (base + c * SEG) + lane * SEG

    zeros16 = jnp.zeros((LANES,), jnp.float32)
    for k in range(SPREAD // LANES):
        fold_v[pl.ds(k * LANES, LANES)] = zeros16
    pltpu.sync_copy(fold_v, shared.at[pl.ds(base, SPREAD)])
    for r in range(ROWS):
        pltpu.sync_copy(val_v.at[r], shared.at[idx2_v.at[r]], add=True)

    pltpu.sync_copy(shared.at[pl.ds(base, SPREAD)], fold_v)
    tot = zeros16
    for k in range(SPREAD // LANES):
        tot = tot + fold_v[pl.ds(k * LANES, LANES)]
    vec_v[...] = tot

    pltpu.sync_copy(vec_v, stage.at[pl.ds(sid * SEG, SEG)])
    plsc.subcore_barrier()

    @pl.when(sid == 0)
    def _():
        pltpu.sync_copy(stage, all_v)
        tot2 = zeros16
        for s in range(NS):
            tot2 = tot2 + all_v[pl.ds(s * SEG, SEG)]
        vec_v[...] = tot2
        pltpu.sync_copy(vec_v, out_hbm.at[cid])


@functools.cache
def _segsum():
    return pl.kernel(
        _segsum_body,
        out_type=jax.ShapeDtypeStruct((NC, SEG), jnp.float32),
        mesh=plsc.VectorSubcoreMesh(
            core_axis_name="c", subcore_axis_name="s",
            num_cores=NC, num_subcores=NS,
        ),
        scratch_types=[
            pltpu.VMEM((ROWS, 128), jnp.float32),
            pltpu.VMEM((CHUNK,), jnp.int32),
            pltpu.VMEM((ROWS, 128), jnp.int32),
            pltpu.VMEM((SPREAD,), jnp.float32),
            pltpu.VMEM((LANES,), jnp.float32),
            pltpu.VMEM((NS * SEG,), jnp.float32),
            pltpu.VMEM_SHARED((NS * SPREAD,), jnp.float32),
            pltpu.VMEM_SHARED((NS * SEG,), jnp.float32),
        ],
    )


@jax.jit
def kernel(h, v, atomic_numbers, pos, batch, W1, b1, W2, b2):
    s = _mlp(h, W1, b1, W2, b2)
    partial = _segsum()(s, batch)
    return (partial[0] + partial[1]).reshape(SEG, 1)


# speedup vs baseline: 1.0444x; 1.0444x over previous
import functools

import jax
import jax.numpy as jnp
from jax import lax
from jax.experimental import pallas as pl
from jax.experimental.pallas import tpu as pltpu
from jax.experimental.pallas import tpu_sc as plsc

N_ATOMS = 32768
HIDDEN = 256
HALF = 128
SEG = 16

NC = 2
NS = 16
NW = NC * NS
CHUNK = N_ATOMS // NW
LANES = 16

BLK = 8192


def _mlp_body(h_ref, w1_ref, b1_ref, w2_ref, b2_ref, s_ref):
    w1t = jnp.transpose(w1_ref[...], (1, 0))
    x = jnp.dot(h_ref[...], w1t, preferred_element_type=jnp.float32)
    x = x + b1_ref[...]
    x = x * jax.nn.sigmoid(x)
    s = jnp.sum(x * w2_ref[...], axis=1)
    s_ref[...] = s.reshape(BLK // 128, 128) + b2_ref[...]


def _mlp(h, W1, b1, W2, b2):
    grid = N_ATOMS // BLK
    return pl.pallas_call(
        _mlp_body,
        grid=(grid,),
        in_specs=[
            pl.BlockSpec((BLK, HIDDEN), lambda g: (g, 0)),
            pl.BlockSpec((HALF, HIDDEN), lambda g: (0, 0)),
            pl.BlockSpec((1, HALF), lambda g: (0, 0)),
            pl.BlockSpec((1, HALF), lambda g: (0, 0)),
            pl.BlockSpec((1, 1), lambda g: (0, 0)),
        ],
        out_specs=pl.BlockSpec((BLK // 128, 128), lambda g: (g, 0)),
        out_shape=jax.ShapeDtypeStruct((N_ATOMS // 128, 128), jnp.float32),
        compiler_params=pltpu.CompilerParams(
            dimension_semantics=("arbitrary",),
        ),
    )(h, W1, b1.reshape(1, HALF),
      W2.reshape(1, HALF), b2.reshape(1, 1))


ROWS = CHUNK // 128
SPREAD = 128 * SEG


def _segsum_body(s_hbm, b_hbm, out_hbm, val_v, idx_v, idx2_v, fold_v, vec_v,
                 all_v, shared, stage):
    cid = lax.axis_index("c")
    sid = lax.axis_index("s")
    wid = cid * NS + sid
    pltpu.sync_copy(s_hbm.at[pl.ds(wid * ROWS, ROWS)], val_v)
    pltpu.sync_copy(b_hbm.at[pl.ds(wid * CHUNK, CHUNK)], idx_v)

    lane = lax.iota(jnp.int32, LANES)
    base = sid * SPREAD
    for i in range(CHUNK // LANES):
        r, c = divmod(i * LANES, 128)
        d = idx_v[pl.ds(i * LANES, LANES)]
        idx2_v[r, pl.ds(c, LANES)] = d + (base + c * SEG) + lane * SEG

    zeros16 = jnp.zeros((LANES,), jnp.float32)
    for k in range(SPREAD // LANES):
        fold_v[pl.ds(k * LANES, LANES)] = zeros16
    pltpu.sync_copy(fold_v, shared.at[pl.ds(base, SPREAD)])
    for r in range(ROWS):
        pltpu.sync_copy(val_v.at[r], shared.at[idx2_v.at[r]], add=True)

    pltpu.sync_copy(shared.at[pl.ds(base, SPREAD)], fold_v)
    tot = zeros16
    for k in range(SPREAD // LANES):
        tot = tot + fold_v[pl.ds(k * LANES, LANES)]
    vec_v[...] = tot

    pltpu.sync_copy(vec_v, stage.at[pl.ds(sid * SEG, SEG)])
    plsc.subcore_barrier()

    @pl.when(sid == 0)
    def _():
        pltpu.sync_copy(stage, all_v)
        tot2 = zeros16
        for s in range(NS):
            tot2 = tot2 + all_v[pl.ds(s * SEG, SEG)]
        vec_v[...] = tot2
        pltpu.sync_copy(vec_v, out_hbm.at[cid])


@functools.cache
def _segsum():
    return pl.kernel(
        _segsum_body,
        out_type=jax.ShapeDtypeStruct((NC, SEG), jnp.float32),
        mesh=plsc.VectorSubcoreMesh(
            core_axis_name="c", subcore_axis_name="s",
            num_cores=NC, num_subcores=NS,
        ),
        scratch_types=[
            pltpu.VMEM((ROWS, 128), jnp.float32),
            pltpu.VMEM((CHUNK,), jnp.int32),
            pltpu.VMEM((ROWS, 128), jnp.int32),
            pltpu.VMEM((SPREAD,), jnp.float32),
            pltpu.VMEM((LANES,), jnp.float32),
            pltpu.VMEM((NS * SEG,), jnp.float32),
            pltpu.VMEM_SHARED((NS * SPREAD,), jnp.float32),
            pltpu.VMEM_SHARED((NS * SEG,), jnp.float32),
        ],
    )


@jax.jit
def kernel(h, v, atomic_numbers, pos, batch, W1, b1, W2, b2):
    s = _mlp(h, W1, b1, W2, b2)
    partial = _segsum()(s, batch)
    return (partial[0] + partial[1]).reshape(SEG, 1)
